# Initial kernel scaffold; baseline (speedup 1.0000x reference)
import jax
import jax.numpy as jnp
from jax.experimental import pallas as pl


def kernel(x, graph_edge_index, pos_edge_index, neg_edge_index, W1, bn_gamma, bn_beta, W2, Wp1, bp1, Wp2, bp2, Wp3, bp3):
    raise NotImplementedError("write your pallas kernel here")


# speedup vs baseline: 4.8053x; 4.8053x over previous
import functools

import jax
import jax.numpy as jnp
from jax import lax
from jax.experimental import pallas as pl
from jax.experimental.pallas import tpu as pltpu
from jax.experimental.pallas import tpu_sc as plsc

N = 10000
D = 128
ALPHA = 0.5
EPS = 1e-5
CHUNK = 128
GROUP = 2
NSLOT = N + 240


def _mlp_body(x_ref, w1_ref, g_ref, b_ref, w2_ref, o_ref):
    t = jnp.dot(x_ref[...], w1_ref[...], preferred_element_type=jnp.float32)
    mean = jnp.mean(t, axis=0, keepdims=True)
    var = jnp.mean((t - mean) ** 2, axis=0, keepdims=True)
    tn = (t - mean) * lax.rsqrt(var + EPS) * g_ref[...] + b_ref[...]
    o_ref[...] = jnp.dot(jnp.maximum(tn, 0.0), w2_ref[...],
                         preferred_element_type=jnp.float32)


def _tc_mlp(x, W1, bn_gamma, bn_beta, W2):
    return pl.pallas_call(
        _mlp_body,
        out_shape=jax.ShapeDtypeStruct((N, D), jnp.float32),
    )(x, W1, bn_gamma.reshape(1, D), bn_beta.reshape(1, D), W2)


def _dinv_of(degp_ref):
    deg = degp_ref[0, :N, 0:1] + degp_ref[1, :N, 0:1]
    return lax.rsqrt(jnp.maximum(deg, 1.0))


def _scale_body(degp_ref, h0_ref, o_ref):
    o_ref[...] = h0_ref[...] * _dinv_of(degp_ref)


def _tc_scale(degp, h0):
    return pl.pallas_call(
        _scale_body,
        out_shape=jax.ShapeDtypeStruct((N, D), jnp.float32),
    )(degp, h0)


def _combine_body(degp_ref, aggp_ref, h0_ref, o_ref, *, scale_out):
    dinv = _dinv_of(degp_ref)
    agg = aggp_ref[0, :N, :] + aggp_ref[1, :N, :]
    h = (1.0 - ALPHA) * (agg * dinv) + ALPHA * h0_ref[...]
    o_ref[...] = h * dinv if scale_out else h


def _tc_combine(degp, aggp, h0, scale_out):
    return pl.pallas_call(
        functools.partial(_combine_body, scale_out=scale_out),
        out_shape=jax.ShapeDtypeStruct((N, D), jnp.float32),
    )(degp, aggp, h0)


def _predict_body(a_ref, b_ref, w1_ref, b1_ref, w2_ref, b2_ref, w3_ref,
                  b3_ref, o_ref):
    hp = a_ref[...] * b_ref[...]
    z = jnp.dot(hp, w1_ref[...], preferred_element_type=jnp.float32)
    z = jnp.maximum(z + b1_ref[...], 0.0)
    z = jnp.dot(z, w2_ref[...], preferred_element_type=jnp.float32)
    z = jnp.maximum(z + b2_ref[...], 0.0)
    o_ref[...] = (jnp.dot(z, w3_ref[...], preferred_element_type=jnp.float32)
                  + b3_ref[...])


def _tc_predict(ha, hb, Wp1, bp1, Wp2, bp2, Wp3, bp3):
    rows = ha.shape[0]
    br = 8192
    grid = rows // br
    full = lambda i: (0, 0)
    return pl.pallas_call(
        _predict_body,
        grid=(grid,),
        in_specs=[
            pl.BlockSpec((br, D), lambda i: (i, 0)),
            pl.BlockSpec((br, D), lambda i: (i, 0)),
            pl.BlockSpec((D, D), full),
            pl.BlockSpec((1, D), full),
            pl.BlockSpec((D, D), full),
            pl.BlockSpec((1, D), full),
            pl.BlockSpec((D, 1), full),
            pl.BlockSpec((1, 1), full),
        ],
        out_specs=pl.BlockSpec((br, 1), lambda i: (i, 0)),
        out_shape=jax.ShapeDtypeStruct((rows, 1), jnp.float32),
    )(ha, hb, Wp1, bp1.reshape(1, D), Wp2, bp2.reshape(1, D), Wp3,
      bp3.reshape(1, 1))


def _sc_mesh():
    return plsc.VectorSubcoreMesh(core_axis_name="c", subcore_axis_name="s")


def _sc_degree(dst3, zerosd, onesd, nc, ns, cpt):
    rows_per_tile = NSLOT // ns
    wb = rows_per_tile // CHUNK
    ngrp = 4
    groups = cpt // ngrp

    @functools.partial(
        pl.kernel,
        out_type=jax.ShapeDtypeStruct((nc, NSLOT, D), jnp.float32),
        mesh=_sc_mesh(),
        scratch_types=[
            pltpu.VMEM_SHARED((NSLOT, D), jnp.float32),
            pltpu.VMEM((cpt, CHUNK), jnp.int32),
            pltpu.VMEM((CHUNK, D), jnp.float32),
        ] + [pltpu.SemaphoreType.DMA for _ in range(ngrp)],
    )
    def k(dst_hbm, zeros_hbm, ones_hbm, out_hbm, shared, dstbuf, ones_v,
          *sems):
        cid = lax.axis_index("c")
        sid = lax.axis_index("s")
        wid = sid * nc + cid
        base = sid * rows_per_tile
        pltpu.sync_copy(zeros_hbm, ones_v)
        for t in range(wb):
            pltpu.sync_copy(ones_v, shared.at[pl.ds(base + t * CHUNK, CHUNK)])
        pltpu.sync_copy(ones_hbm, ones_v)
        pltpu.sync_copy(dst_hbm.at[wid], dstbuf)
        plsc.subcore_barrier()

        def group(g, _):
            j0 = g * ngrp
            cps = [
                pltpu.async_copy(ones_v, shared.at[dstbuf.at[j0 + b]],
                                 sems[b], add=True)
                for b in range(ngrp)
            ]
            for cp in cps:
                cp.wait()
            return 0

        lax.fori_loop(0, groups, group, 0)
        plsc.subcore_barrier()
        for t in range(wb):
            pltpu.sync_copy(shared.at[pl.ds(base + t * CHUNK, CHUNK)], ones_v)
            pltpu.sync_copy(ones_v,
                            out_hbm.at[cid, pl.ds(base + t * CHUNK, CHUNK)])

    return k(dst3, zerosd, onesd)


def _sc_scatter(hs, src3, dst3, zerosd, nc, ns, cpt):
    rows_per_tile = NSLOT // ns
    wb = rows_per_tile // CHUNK
    hcpt = cpt // 2
    groups = hcpt // GROUP

    @functools.partial(
        pl.kernel,
        out_type=jax.ShapeDtypeStruct((nc, NSLOT, D), jnp.float32),
        mesh=_sc_mesh(),
        scratch_types=[
            pltpu.VMEM_SHARED((NSLOT, D), jnp.float32),
            pltpu.VMEM((hcpt, CHUNK), jnp.int32),
            pltpu.VMEM((hcpt, CHUNK), jnp.int32),
        ] + [pltpu.VMEM((CHUNK, D), jnp.float32) for _ in range(GROUP)]
          + [pltpu.SemaphoreType.DMA for _ in range(2 * GROUP)],
    )
    def k(hs_hbm, src_hbm, dst_hbm, zeros_hbm, out_hbm, shared, srcbuf,
          dstbuf, *rest):
        bufs = rest[:GROUP]
        gsems = rest[GROUP:2 * GROUP]
        ssems = rest[2 * GROUP:]
        cid = lax.axis_index("c")
        sid = lax.axis_index("s")
        wid = sid * nc + cid
        base = sid * rows_per_tile
        pltpu.sync_copy(zeros_hbm, bufs[0])
        for t in range(wb):
            pltpu.sync_copy(bufs[0], shared.at[pl.ds(base + t * CHUNK, CHUNK)])

        def group(g, _):
            j0 = g * GROUP
            gcps = [
                pltpu.async_copy(hs_hbm.at[srcbuf.at[j0 + b]], bufs[b],
                                 gsems[b])
                for b in range(GROUP)
            ]
            scps = []
            for b in range(GROUP):
                gcps[b].wait()
                scps.append(
                    pltpu.async_copy(bufs[b], shared.at[dstbuf.at[j0 + b]],
                                     ssems[b], add=True))
            for b in range(GROUP):
                scps[b].wait()
            return 0

        for half in range(2):
            pltpu.sync_copy(src_hbm.at[wid, pl.ds(half * hcpt, hcpt)], srcbuf)
            pltpu.sync_copy(dst_hbm.at[wid, pl.ds(half * hcpt, hcpt)], dstbuf)
            if half == 0:
                plsc.subcore_barrier()
            lax.fori_loop(0, groups, group, 0)

        plsc.subcore_barrier()
        for t in range(wb):
            pltpu.sync_copy(shared.at[pl.ds(base + t * CHUNK, CHUNK)], bufs[0])
            pltpu.sync_copy(bufs[0],
                            out_hbm.at[cid, pl.ds(base + t * CHUNK, CHUNK)])

    return k(hs, src3, dst3, zerosd)


def _sc_pair_gather(h, idxa3, idxb3, nc, ns, cpt):
    rows = idxa3.shape[0] * cpt * CHUNK
    rows_per_tile = cpt * CHUNK

    @functools.partial(
        pl.kernel,
        out_type=[
            jax.ShapeDtypeStruct((rows, D), jnp.float32),
            jax.ShapeDtypeStruct((rows, D), jnp.float32),
        ],
        mesh=_sc_mesh(),
        scratch_types=[
            pltpu.VMEM((cpt, CHUNK), jnp.int32),
            pltpu.VMEM((cpt, CHUNK), jnp.int32),
            pltpu.VMEM((CHUNK, D), jnp.float32),
            pltpu.VMEM((CHUNK, D), jnp.float32),
            pltpu.SemaphoreType.DMA,
            pltpu.SemaphoreType.DMA,
        ],
    )
    def k(h_hbm, ia_hbm, ib_hbm, oa_hbm, ob_hbm, abuf, bbuf, rowa, rowb,
          sema, semb):
        cid = lax.axis_index("c")
        sid = lax.axis_index("s")
        wid = sid * nc + cid
        base = wid * rows_per_tile
        pltpu.sync_copy(ia_hbm.at[wid], abuf)
        pltpu.sync_copy(ib_hbm.at[wid], bbuf)

        def chunk(j, _):
            ca = pltpu.async_copy(h_hbm.at[abuf.at[j]], rowa, sema)
            cb = pltpu.async_copy(h_hbm.at[bbuf.at[j]], rowb, semb)
            ca.wait()
            cb.wait()
            wa = pltpu.async_copy(rowa, oa_hbm.at[pl.ds(base + j * CHUNK,
                                                        CHUNK)], sema)
            wb = pltpu.async_copy(rowb, ob_hbm.at[pl.ds(base + j * CHUNK,
                                                        CHUNK)], semb)
            wa.wait()
            wb.wait()
            return 0

        lax.fori_loop(0, cpt, chunk, 0)

    return k(h, idxa3, idxb3)


def kernel(x, graph_edge_index, pos_edge_index, neg_edge_index, W1, bn_gamma,
           bn_beta, W2, Wp1, bp1, Wp2, bp2, Wp3, bp3):
    info = plsc.get_sparse_core_info()
    nc, ns = info.num_cores, info.num_subcores
    nw = nc * ns

    e = graph_edge_index.shape[1]
    cpt = -(-e // (nw * CHUNK))
    cpt = -(-cpt // (2 * GROUP)) * 2 * GROUP
    e_pad = nw * cpt * CHUNK
    npad = e_pad - e
    src_pad = jnp.concatenate(
        [graph_edge_index[0], jnp.zeros((npad,), jnp.int32)])
    dst_pad = jnp.concatenate(
        [graph_edge_index[1],
         N + (jnp.arange(npad, dtype=jnp.int32) % (NSLOT - N))])
    src3 = src_pad.reshape(nw, cpt, CHUNK)
    dst3 = dst_pad.reshape(nw, cpt, CHUNK)

    zerosd = jnp.zeros((CHUNK, D), jnp.float32)
    onesd = jnp.ones((CHUNK, D), jnp.float32)

    h0 = _tc_mlp(x, W1, bn_gamma, bn_beta, W2)
    degp = _sc_degree(dst3, zerosd, onesd, nc, ns, cpt)

    hs = _tc_scale(degp, h0)
    aggp = _sc_scatter(hs, src3, dst3, zerosd, nc, ns, cpt)
    hs = _tc_combine(degp, aggp, h0, scale_out=True)
    aggp = _sc_scatter(hs, src3, dst3, zerosd, nc, ns, cpt)
    h = _tc_combine(degp, aggp, h0, scale_out=False)

    ep = pos_edge_index.shape[1]
    idxa = jnp.concatenate([pos_edge_index[0], neg_edge_index[0]])
    idxb = jnp.concatenate([pos_edge_index[1], neg_edge_index[1]])
    pcpt = (2 * ep) // (nw * CHUNK)
    idxa3 = idxa.reshape(nw, pcpt, CHUNK)
    idxb3 = idxb.reshape(nw, pcpt, CHUNK)
    ha, hb = _sc_pair_gather(h, idxa3, idxb3, nc, ns, pcpt)

    out = _tc_predict(ha, hb, Wp1, bp1, Wp2, bp2, Wp3, bp3)
    return (out[:ep], out[ep:])
